# Initial kernel scaffold; baseline (speedup 1.0000x reference)
import jax
import jax.numpy as jnp
from jax.experimental import pallas as pl


def kernel(x, ew1, eb1, ew2, eb2, ew3, eb3, ew4, eb4, dw1, db1, dw2, db2, dw3, db3, dw4, db4, embed):
    raise NotImplementedError("write your pallas kernel here")


# speedup vs baseline: 1.4356x; 1.4356x over previous
import functools

import jax
import jax.numpy as jnp
from jax.experimental import pallas as pl
from jax.experimental.pallas import tpu as pltpu

_INTERPRET = False
_F32 = jnp.float32


def _tap_w(w_oihw):
    o, i, kh, kw = w_oihw.shape
    return jnp.transpose(w_oihw, (2, 3, 1, 0)).reshape(kh * kw, i, o)


def _tap_wT(w_iohw):
    i, o, kh, kw = w_iohw.shape
    return jnp.transpose(w_iohw, (2, 3, 0, 1)).reshape(kh * kw, i, o)


def _conv_s1(xp, w9, b2, act, br):
    n, hp, wp, ci = xp.shape
    h, w, co = hp - 2, wp - 2, w9.shape[2]
    nr = h // br

    def body(x_ref, w_ref, b_ref, o_ref):
        acc = jnp.zeros((br * w, co), _F32)
        for dy in range(3):
            for dx in range(3):
                a = x_ref[0, dy:dy + br, dx:dx + w, :].reshape(br * w, ci)
                acc = acc + jnp.dot(a, w_ref[dy * 3 + dx],
                                    preferred_element_type=_F32)
        v = acc + b_ref[0:1, :]
        if act:
            v = jnp.maximum(v, 0.0)
        o_ref[0] = v.reshape(br, w, co)

    return pl.pallas_call(
        body,
        grid=(n, nr),
        in_specs=[pl.BlockSpec((pl.Element(1), pl.Element(br + 2),
                                pl.Element(wp), pl.Element(ci)),
                               lambda i, r: (i, r * br, 0, 0)),
                  pl.BlockSpec((9, ci, co), lambda i, r: (0, 0, 0)),
                  pl.BlockSpec((1, co), lambda i, r: (0, 0))],
        out_specs=pl.BlockSpec((1, br, w, co), lambda i, r: (i, r, 0, 0)),
        out_shape=jax.ShapeDtypeStruct((n, h, w, co), _F32),
        interpret=_INTERPRET,
    )(xp, w9, b2)


def _conv_s2(x5, w9, b2, br):
    n, u, _, v, ci2 = x5.shape
    ci = ci2 // 2
    ho, wo, co = u - 1, v - 1, w9.shape[2]
    nr = ho // br

    def body(x_ref, w_ref, b_ref, o_ref):
        acc = jnp.zeros((br * wo, co), _F32)
        for dy in range(3):
            for dx in range(3):
                di, p = dy // 2, dy % 2
                dj, c0 = dx // 2, (dx % 2) * ci
                a = x_ref[0, di:di + br, p, dj:dj + wo,
                          c0:c0 + ci].reshape(br * wo, ci)
                acc = acc + jnp.dot(a, w_ref[dy * 3 + dx],
                                    preferred_element_type=_F32)
        v2 = jnp.maximum(acc + b_ref[0:1, :], 0.0)
        o_ref[0] = v2.reshape(br, wo, co)

    return pl.pallas_call(
        body,
        grid=(n, nr),
        in_specs=[pl.BlockSpec((pl.Element(1), pl.Element(br + 1),
                                pl.Element(2), pl.Element(v),
                                pl.Element(ci2)),
                               lambda i, r: (i, r * br, 0, 0, 0)),
                  pl.BlockSpec((9, ci, co), lambda i, r: (0, 0, 0)),
                  pl.BlockSpec((1, co), lambda i, r: (0, 0))],
        out_specs=pl.BlockSpec((1, br, wo, co), lambda i, r: (i, r, 0, 0)),
        out_shape=jax.ShapeDtypeStruct((n, ho, wo, co), _F32),
        interpret=_INTERPRET,
    )(x5, w9, b2)


_CT_TAPS = {
    (0, 0): ((0, 0, 4),),
    (0, 1): ((0, 0, 5), (0, 1, 3)),
    (1, 0): ((0, 0, 7), (1, 0, 1)),
    (1, 1): ((0, 0, 8), (0, 1, 6), (1, 0, 2), (1, 1, 0)),
}


def _convt_s2(xp, w9, b2, br):
    n, hp1, wp1, ci = xp.shape
    hi, wi, co = hp1 - 1, wp1 - 1, w9.shape[2]
    nr = hi // br

    def body(x_ref, w_ref, b_ref, o_ref):
        for (a, b), taps in _CT_TAPS.items():
            acc = jnp.zeros((br * wi, co), _F32)
            for (si, sj, t) in taps:
                av = x_ref[0, si:si + br, sj:sj + wi, :].reshape(br * wi, ci)
                acc = acc + jnp.dot(av, w_ref[t], preferred_element_type=_F32)
            r = jnp.maximum(acc + b_ref[0:1, :], 0.0)
            o_ref[0, :, a, :, b * co:(b + 1) * co] = r.reshape(br, wi, co)

    return pl.pallas_call(
        body,
        grid=(n, nr),
        in_specs=[pl.BlockSpec((pl.Element(1), pl.Element(br + 1),
                                pl.Element(wp1), pl.Element(ci)),
                               lambda i, r: (i, r * br, 0, 0)),
                  pl.BlockSpec((9, ci, co), lambda i, r: (0, 0, 0)),
                  pl.BlockSpec((1, co), lambda i, r: (0, 0))],
        out_specs=pl.BlockSpec((1, br, 2, wi, 2 * co),
                               lambda i, r: (i, r, 0, 0, 0)),
        out_shape=jax.ShapeDtypeStruct((n, hi, 2, wi, 2 * co), _F32),
        interpret=_INTERPRET,
    )(xp, w9, b2)


def _vq(flat, embed):
    m, d = flat.shape
    k = embed.shape[0]
    nblk = 8
    blk = m // nblk

    def body(z_ref, e_ref, zq_ref, sse_ref):
        i = pl.program_id(0)
        z = z_ref[...]
        e = e_ref[...]
        zd = jax.lax.dot_general(z, e, (((1,), (1,)), ((), ())),
                                 preferred_element_type=_F32)
        e2 = jax.lax.dot_general(jnp.ones((1, d), _F32), e * e,
                                 (((1,), (1,)), ((), ())),
                                 preferred_element_type=_F32)
        z2 = jnp.sum(z * z, axis=1, keepdims=True)
        dist = z2 - 2.0 * zd + e2
        mn = jnp.min(dist, axis=1, keepdims=True)
        li = jax.lax.broadcasted_iota(jnp.int32, (blk, k), 1)
        idx = jnp.min(jnp.where(dist == mn, li, k), axis=1, keepdims=True)
        oh = (li == idx).astype(_F32)
        zq = jnp.dot(oh, e, preferred_element_type=_F32)
        zq_ref[...] = zq
        s = jnp.sum((zq - z) ** 2)

        @pl.when(i == 0)
        def _():
            sse_ref[0, 0] = 0.0

        sse_ref[0, 0] += s

    return pl.pallas_call(
        body,
        grid=(nblk,),
        in_specs=[pl.BlockSpec((blk, d), lambda i: (i, 0)),
                  pl.BlockSpec((k, d), lambda i: (0, 0))],
        out_specs=[pl.BlockSpec((blk, d), lambda i: (i, 0)),
                   pl.BlockSpec((1, 1), lambda i: (0, 0),
                                memory_space=pltpu.SMEM)],
        out_shape=[jax.ShapeDtypeStruct((m, d), _F32),
                   jax.ShapeDtypeStruct((1, 1), _F32)],
        interpret=_INTERPRET,
    )(flat, embed)


def _convt_s1_loss(xp, w9, b2, target, br):
    n, hp, wp, ci = xp.shape
    h, w, co = hp - 2, wp - 2, w9.shape[2]
    nr = h // br

    def body(x_ref, w_ref, b_ref, t_ref, o_ref, sse_ref):
        first = (pl.program_id(0) == 0) & (pl.program_id(1) == 0)
        acc = jnp.zeros((br * w, co), _F32)
        for dy in range(3):
            for dx in range(3):
                a = x_ref[0, dy:dy + br, dx:dx + w, :].reshape(br * w, ci)
                acc = acc + jnp.dot(a, w_ref[dy * 3 + dx],
                                    preferred_element_type=_F32)
        r = (acc + b_ref[0:1, :]).reshape(br, w, co)
        o_ref[0] = r
        dlt = r - t_ref[0]
        s = jnp.sum(dlt * dlt)

        @pl.when(first)
        def _():
            sse_ref[0, 0] = 0.0

        sse_ref[0, 0] += s

    return pl.pallas_call(
        body,
        grid=(n, nr),
        in_specs=[pl.BlockSpec((pl.Element(1), pl.Element(br + 2),
                                pl.Element(wp), pl.Element(ci)),
                               lambda i, r: (i, r * br, 0, 0)),
                  pl.BlockSpec((9, ci, co), lambda i, r: (0, 0, 0)),
                  pl.BlockSpec((1, co), lambda i, r: (0, 0)),
                  pl.BlockSpec((1, br, w, co), lambda i, r: (i, r, 0, 0))],
        out_specs=[pl.BlockSpec((1, br, w, co), lambda i, r: (i, r, 0, 0)),
                   pl.BlockSpec((1, 1), lambda i, r: (0, 0),
                                memory_space=pltpu.SMEM)],
        out_shape=[jax.ShapeDtypeStruct((n, h, w, co), _F32),
                   jax.ShapeDtypeStruct((1, 1), _F32)],
        interpret=_INTERPRET,
    )(xp, w9, b2, target)


def _pad1(a):
    return jnp.pad(a, ((0, 0), (1, 1), (1, 1), (0, 0)))


def kernel(x, ew1, eb1, ew2, eb2, ew3, eb3, ew4, eb4,
           dw1, db1, dw2, db2, dw3, db3, dw4, db4, embed):
    n = x.shape[0]
    xh = jnp.transpose(x, (0, 2, 3, 1))

    h1 = _conv_s1(_pad1(xh), _tap_w(ew1), eb1[None, :], True, 28)
    h1p = _pad1(h1).reshape(n, 113, 2, 113, 32)
    h2 = _conv_s2(h1p, _tap_w(ew2), eb2[None, :], 28)
    h2p = _pad1(h2).reshape(n, 57, 2, 57, 64)
    h3 = _conv_s2(h2p, _tap_w(ew3), eb3[None, :], 28)
    h3p = _pad1(h3).reshape(n, 29, 2, 29, 128)
    z = _conv_s2(h3p, _tap_w(ew4), eb4[None, :], 28)

    flat = z.reshape(-1, 128)
    zq_flat, sse_vq = _vq(flat, embed)
    diff = 2.0 * sse_vq[0, 0] / flat.size
    zq = zq_flat.reshape(n, 28, 28, 128)

    zqp = jnp.pad(zq, ((0, 0), (0, 1), (0, 1), (0, 0)))
    d1 = _convt_s2(zqp, _tap_wT(dw1), db1[None, :], 28).reshape(n, 56, 56, 64)
    d1p = jnp.pad(d1, ((0, 0), (0, 1), (0, 1), (0, 0)))
    d2 = _convt_s2(d1p, _tap_wT(dw2), db2[None, :], 28).reshape(n, 112, 112, 32)
    d2p = jnp.pad(d2, ((0, 0), (0, 1), (0, 1), (0, 0)))
    d3 = _convt_s2(d2p, _tap_wT(dw3), db3[None, :], 28).reshape(n, 224, 224, 16)

    w4 = jnp.transpose(jnp.flip(dw4, (2, 3)), (2, 3, 0, 1)).reshape(9, 16, 3)
    x_rec_h, sse_rec = _convt_s1_loss(_pad1(d3), w4, db4[None, :], xh, 28)

    x_rec = jnp.transpose(x_rec_h, (0, 3, 1, 2))
    loss = sse_rec[0, 0] / x_rec.size + 0.25 * diff
    return (x_rec, loss)


# speedup vs baseline: 4.2036x; 2.9282x over previous
import jax
import jax.numpy as jnp
import numpy as np
from jax.experimental import pallas as pl
from jax.experimental.pallas import tpu as pltpu

_INTERPRET = False
_F32 = jnp.float32


def _tap_wT(w_iohw):
    i, o, kh, kw = w_iohw.shape
    return jnp.transpose(w_iohw, (2, 3, 0, 1)).reshape(kh * kw, i, o)


def _gfold_w(w_cd, g):
    ci, co = w_cd.shape[2], w_cd.shape[3]
    blocks = []
    for dy in range(3):
        for s in range(3):
            b = jnp.zeros((g * ci, g * co), _F32)
            for dx in range(3):
                k = dx - 1 + (1 - s) * g
                if -g < k < g:
                    b = b + jnp.kron(jnp.eye(g, k=-k, dtype=_F32),
                                     w_cd[dy, dx])
            blocks.append(b)
    return jnp.stack(blocks)


def _gfold_s2_w(w_cd, g):
    g = int(g)
    e1 = np.zeros((2 * g, g), np.float32)
    e2 = np.zeros((2 * g, g), np.float32)
    e0 = np.zeros((2 * g, g), np.float32)
    el = np.zeros((2 * g, g), np.float32)
    for q in range(g):
        e1[2 * q, q] = 1
        e2[2 * q + 1, q] = 1
        if q + 1 < g:
            e0[2 * q + 1, q + 1] = 1
    el[2 * g - 1, 0] = 1
    mats = []
    for dy in range(3):
        mats.append(jnp.kron(jnp.asarray(el), w_cd[dy, 0]))
        mats.append(jnp.kron(jnp.asarray(e1), w_cd[dy, 1])
                    + jnp.kron(jnp.asarray(e2), w_cd[dy, 2])
                    + jnp.kron(jnp.asarray(e0), w_cd[dy, 0]))
    return jnp.stack(mats)


_CT_TAPS = {
    (0, 0): ((0, 0, 4),),
    (0, 1): ((0, 0, 5), (0, 1, 3)),
    (1, 0): ((0, 0, 7), (1, 0, 1)),
    (1, 1): ((0, 0, 8), (0, 1, 6), (1, 0, 2), (1, 1, 0)),
}
_CT_SPECS = ((0, 0, 0), (0, 0, 1), (1, 0, 0), (1, 0, 1), (1, 1, 0), (1, 1, 1))


def _gfoldt_w(w9, gi):
    gi = int(gi)

    def sel(sj, b):
        s = np.zeros((gi, 2 * gi), np.float32)
        for q in range(gi):
            if sj == 0:
                s[q, 2 * q + b] = 1
            elif q + 1 < gi:
                s[q + 1, 2 * q + b] = 1
        return s

    def selc(b):
        s = np.zeros((gi, 2 * gi), np.float32)
        s[0, 2 * (gi - 1) + b] = 1
        return s

    mats = []
    for a in (0, 1):
        for si in ((0,) if a == 0 else (0, 1)):
            c = jnp.zeros((gi * w9.shape[1], 2 * gi * w9.shape[2]), _F32)
            r = jnp.zeros((gi * w9.shape[1], 2 * gi * w9.shape[2]), _F32)
            for b in (0, 1):
                for (si2, sj, t) in _CT_TAPS[(a, b)]:
                    if si2 != si:
                        continue
                    if sj == 0:
                        c = c + jnp.kron(jnp.asarray(sel(0, b)), w9[t])
                    else:
                        c = c + jnp.kron(jnp.asarray(sel(1, b)), w9[t])
                        r = r + jnp.kron(jnp.asarray(selc(b)), w9[t])
            mats += [c, r]
    return jnp.stack(mats)


def _zero_borders(o_ref, uo):
    o_ref[0, 0] = jnp.zeros_like(o_ref[0, 0])
    for u in range(uo + 1, o_ref.shape[1]):
        o_ref[0, u] = jnp.zeros_like(o_ref[0, u])
    o_ref[0, :, :, 0, :] = jnp.zeros_like(o_ref[0, :, :, 0, :])
    o_ref[0, :, :, 29, :] = jnp.zeros_like(o_ref[0, :, :, 29, :])


def _conv1(xg, w9, b2):
    n = xg.shape[0]

    def body(x_ref, w_ref, b_ref, o_ref):
        acc = jnp.zeros((224 * 28, 128), _F32)
        for dy in range(3):
            for s in range(3):
                a = x_ref[0, dy:dy + 224, s:s + 28, :].reshape(224 * 28, 24)
                acc = acc + jnp.dot(a, w_ref[dy * 3 + s],
                                    preferred_element_type=_F32)
        v = jnp.maximum(acc + b_ref[0:1, :], 0.0)
        o_ref[0, 1:113, :, 1:29, :] = v.reshape(112, 2, 28, 128)
        _zero_borders(o_ref, 112)

    return pl.pallas_call(
        body,
        grid=(n,),
        in_specs=[pl.BlockSpec((1, 226, 30, 24), lambda i: (i, 0, 0, 0)),
                  pl.BlockSpec((9, 24, 128), lambda i: (0, 0, 0)),
                  pl.BlockSpec((1, 128), lambda i: (0, 0))],
        out_specs=pl.BlockSpec((1, 114, 2, 30, 128),
                               lambda i: (i, 0, 0, 0, 0)),
        out_shape=jax.ShapeDtypeStruct((n, 114, 2, 30, 128), _F32),
        interpret=_INTERPRET,
    )(xg, w9, b2)


def _conv_s2(x6, w6, b2, fold_out):
    n, u = x6.shape[0], x6.shape[1]
    ho, gco = u - 2, w6.shape[2]

    def body(x_ref, w_ref, b_ref, o_ref):
        acc = jnp.zeros((ho * 28, gco), _F32)
        for dy in range(3):
            di, pu = divmod(dy + 1, 2)
            for s in range(2):
                a = x_ref[0, di:di + ho, pu, s:s + 28, :].reshape(ho * 28, 128)
                acc = acc + jnp.dot(a, w_ref[dy * 2 + s],
                                    preferred_element_type=_F32)
        v = jnp.maximum(acc + b_ref[0:1, :], 0.0)
        if fold_out:
            o_ref[0, 1:1 + ho // 2, :, 1:29, :] = v.reshape(
                ho // 2, 2, 28, gco)
            _zero_borders(o_ref, ho // 2)
        else:
            o_ref[0] = v.reshape(ho, 28, gco)

    if fold_out:
        oshape = (n, ho // 2 + 2, 2, 30, gco)
        ospec = pl.BlockSpec((1,) + oshape[1:], lambda i: (i, 0, 0, 0, 0))
    else:
        oshape = (n, ho, 28, gco)
        ospec = pl.BlockSpec((1,) + oshape[1:], lambda i: (i, 0, 0, 0))
    return pl.pallas_call(
        body,
        grid=(n,),
        in_specs=[pl.BlockSpec((1, u, 2, 30, 128),
                               lambda i: (i, 0, 0, 0, 0)),
                  pl.BlockSpec((6, 128, gco), lambda i: (0, 0, 0)),
                  pl.BlockSpec((1, gco), lambda i: (0, 0))],
        out_specs=ospec,
        out_shape=jax.ShapeDtypeStruct(oshape, _F32),
        interpret=_INTERPRET,
    )(x6, w6, b2)


def _vq(z, embed):
    n = z.shape[0]
    k, d = embed.shape

    def body(z_ref, e_ref, zq_ref, sse_ref):
        i = pl.program_id(0)
        zz = z_ref[0].reshape(784, d)
        e = e_ref[...]
        zd = jax.lax.dot_general(zz, e, (((1,), (1,)), ((), ())),
                                 preferred_element_type=_F32)
        e2 = jax.lax.dot_general(jnp.ones((1, d), _F32), e * e,
                                 (((1,), (1,)), ((), ())),
                                 preferred_element_type=_F32)
        z2 = jnp.sum(zz * zz, axis=1, keepdims=True)
        dist = z2 - 2.0 * zd + e2
        mn = jnp.min(dist, axis=1, keepdims=True)
        li = jax.lax.broadcasted_iota(jnp.int32, (784, k), 1)
        idx = jnp.min(jnp.where(dist == mn, li, k), axis=1, keepdims=True)
        oh = (li == idx).astype(_F32)
        zq = jnp.dot(oh, e, preferred_element_type=_F32)
        zq_ref[0, 0:28, 0:28, :] = zq.reshape(28, 28, d)
        zq_ref[0, 28, :, :] = jnp.zeros((29, d), _F32)
        zq_ref[0, :, 28, :] = jnp.zeros((29, d), _F32)
        s = jnp.sum((zq - zz) ** 2)

        @pl.when(i == 0)
        def _():
            sse_ref[0, 0] = 0.0

        sse_ref[0, 0] += s

    return pl.pallas_call(
        body,
        grid=(n,),
        in_specs=[pl.BlockSpec((1, 28, 28, d), lambda i: (i, 0, 0, 0)),
                  pl.BlockSpec((k, d), lambda i: (0, 0))],
        out_specs=[pl.BlockSpec((1, 29, 29, d), lambda i: (i, 0, 0, 0)),
                   pl.BlockSpec((1, 1), lambda i: (0, 0),
                                memory_space=pltpu.SMEM)],
        out_shape=[jax.ShapeDtypeStruct((n, 29, 29, d), _F32),
                   jax.ShapeDtypeStruct((1, 1), _F32)],
        interpret=_INTERPRET,
    )(z, embed)


def _convt_s2(xg, w6, b2):
    n, hp1, wg1, gci = xg.shape
    hi, wg = hp1 - 1, wg1 - 1
    g2co = w6.shape[2]

    def body(x_ref, w_ref, b_ref, o_ref):
        for a in (0, 1):
            acc = jnp.zeros((hi * wg, g2co), _F32)
            for idx, (aa, si, gofs) in enumerate(_CT_SPECS):
                if aa != a:
                    continue
                v = x_ref[0, si:si + hi, gofs:gofs + wg, :].reshape(
                    hi * wg, gci)
                acc = acc + jnp.dot(v, w_ref[idx],
                                    preferred_element_type=_F32)
            r = jnp.maximum(acc + b_ref[0:1, :], 0.0)
            o_ref[0, :, a, :, :] = r.reshape(hi, wg, g2co)

    return pl.pallas_call(
        body,
        grid=(n,),
        in_specs=[pl.BlockSpec((1, hp1, wg1, gci), lambda i: (i, 0, 0, 0)),
                  pl.BlockSpec((6, gci, g2co), lambda i: (0, 0, 0)),
                  pl.BlockSpec((1, g2co), lambda i: (0, 0))],
        out_specs=pl.BlockSpec((1, hi, 2, wg, g2co),
                               lambda i: (i, 0, 0, 0, 0)),
        out_shape=jax.ShapeDtypeStruct((n, hi, 2, wg, g2co), _F32),
        interpret=_INTERPRET,
    )(xg, w6, b2)


def _convt3(xg, w6, b2):
    n = xg.shape[0]
    hi, wg = 112, 28

    def body(x_ref, w_ref, b_ref, o_ref):
        for a in (0, 1):
            acc = jnp.zeros((hi * wg, 128), _F32)
            for idx, (aa, si, gofs) in enumerate(_CT_SPECS):
                if aa != a:
                    continue
                v = x_ref[0, si:si + hi, gofs:gofs + wg, :].reshape(
                    hi * wg, 128)
                acc = acc + jnp.dot(v, w_ref[idx],
                                    preferred_element_type=_F32)
            r = jnp.maximum(acc + b_ref[0:1, :], 0.0)
            o_ref[0, 1:113, a, 1:29, :] = r.reshape(hi, wg, 128)
        _zero_borders(o_ref, 112)

    return pl.pallas_call(
        body,
        grid=(n,),
        in_specs=[pl.BlockSpec((1, 113, 29, 128), lambda i: (i, 0, 0, 0)),
                  pl.BlockSpec((6, 128, 128), lambda i: (0, 0, 0)),
                  pl.BlockSpec((1, 128), lambda i: (0, 0))],
        out_specs=pl.BlockSpec((1, 115, 2, 30, 128),
                               lambda i: (i, 0, 0, 0, 0)),
        out_shape=jax.ShapeDtypeStruct((n, 115, 2, 30, 128), _F32),
        interpret=_INTERPRET,
    )(xg, w6, b2)


def _conv4t_loss(xf, w9, b2, target):
    n = xf.shape[0]

    def body(x_ref, w_ref, b_ref, t_ref, o_ref, sse_ref):
        i = pl.program_id(0)

        @pl.when(i == 0)
        def _():
            sse_ref[0, 0] = 0.0

        for c in (0, 1):
            acc = jnp.zeros((112 * 28, 24), _F32)
            for dy in range(3):
                q, pu = divmod(c + dy - 1, 2)
                u0 = 1 + q
                for s in range(3):
                    a = x_ref[0, u0:u0 + 112, pu, s:s + 28, :].reshape(
                        112 * 28, 128)
                    acc = acc + jnp.dot(a, w_ref[dy * 3 + s],
                                        preferred_element_type=_F32)
            v = acc + b_ref[0:1, :]
            o_ref[0, :, c, :, :] = v.reshape(112, 28, 24)
            dlt = v - t_ref[0, :, c, :, :].reshape(112 * 28, 24)
            sse_ref[0, 0] += jnp.sum(dlt * dlt)

    return pl.pallas_call(
        body,
        grid=(n,),
        in_specs=[pl.BlockSpec((1, 115, 2, 30, 128),
                               lambda i: (i, 0, 0, 0, 0)),
                  pl.BlockSpec((9, 128, 24), lambda i: (0, 0, 0)),
                  pl.BlockSpec((1, 24), lambda i: (0, 0)),
                  pl.BlockSpec((1, 112, 2, 28, 24),
                               lambda i: (i, 0, 0, 0, 0))],
        out_specs=[pl.BlockSpec((1, 112, 2, 28, 24),
                                lambda i: (i, 0, 0, 0, 0)),
                   pl.BlockSpec((1, 1), lambda i: (0, 0),
                                memory_space=pltpu.SMEM)],
        out_shape=[jax.ShapeDtypeStruct((n, 112, 2, 28, 24), _F32),
                   jax.ShapeDtypeStruct((1, 1), _F32)],
        interpret=_INTERPRET,
    )(xf, w9, b2, target)


def kernel(x, ew1, eb1, ew2, eb2, ew3, eb3, ew4, eb4,
           dw1, db1, dw2, db2, dw3, db3, dw4, db4, embed):
    n = x.shape[0]
    g = 8
    xh = jnp.transpose(x, (0, 2, 3, 1))

    xg = jnp.pad(xh, ((0, 0), (1, 1), (g, g), (0, 0))).reshape(n, 226, 30, 24)
    w1 = _gfold_w(jnp.transpose(ew1, (2, 3, 1, 0)), g)
    a2 = _conv1(xg, w1, jnp.tile(eb1, g)[None, :])
    w2 = _gfold_s2_w(jnp.transpose(ew2, (2, 3, 1, 0)), 4)
    a3 = _conv_s2(a2, w2, jnp.tile(eb2, 4)[None, :], True)
    w3 = _gfold_s2_w(jnp.transpose(ew3, (2, 3, 1, 0)), 2)
    a4 = _conv_s2(a3, w3, jnp.tile(eb3, 2)[None, :], True)
    w4e = _gfold_s2_w(jnp.transpose(ew4, (2, 3, 1, 0)), 1)
    z = _conv_s2(a4, w4e, eb4[None, :], False)

    zqp, sse_vq = _vq(z, embed)
    diff = 2.0 * sse_vq[0, 0] / float(z.size)

    wt1 = _gfoldt_w(_tap_wT(dw1), 1)
    d1 = _convt_s2(zqp, wt1, jnp.tile(db1, 2)[None, :])
    d1 = d1.reshape(n, 56, 56, 64)
    t2in = jnp.pad(d1, ((0, 0), (0, 1), (0, 2), (0, 0))).reshape(
        n, 57, 29, 128)
    wt2 = _gfoldt_w(_tap_wT(dw2), 2)
    d2 = _convt_s2(t2in, wt2, jnp.tile(db2, 4)[None, :])
    d2 = d2.reshape(n, 112, 112, 32)
    t3in = jnp.pad(d2, ((0, 0), (0, 1), (0, 4), (0, 0))).reshape(
        n, 113, 29, 128)
    wt3 = _gfoldt_w(_tap_wT(dw3), 4)
    d3f = _convt3(t3in, wt3, jnp.tile(db3, 8)[None, :])

    w4 = _gfold_w(jnp.transpose(jnp.flip(dw4, (2, 3)), (2, 3, 0, 1)), g)
    xh_f = xh.reshape(n, 112, 2, 28, 24)
    x_rec_f, sse_rec = _conv4t_loss(d3f, w4, jnp.tile(db4, g)[None, :], xh_f)

    x_rec = jnp.transpose(x_rec_f.reshape(n, 224, 224, 3), (0, 3, 1, 2))
    loss = sse_rec[0, 0] / float(x_rec.size) + 0.25 * diff
    return (x_rec, loss)
